# baseline (speedup 1.0000x reference)
import functools

import jax
import jax.numpy as jnp
import numpy as np
from jax import lax
from jax.experimental import pallas as pl
from jax.experimental.pallas import tpu as pltpu
from jax.experimental.pallas import tpu_sc as plsc

N_NODES = 10000
N_EDGES = 320000
D_NODE = 128
D_EDGE = 16
D_GLOBAL = 16
N_GRAPHS = 64
HIDDEN = 128

_NW = 32
_PER_W = N_EDGES // _NW
_CHUNK = 80
_NCHUNK = _PER_W // _CHUNK

_LOG2 = 0.6931471805599453


def _proj_body(node_ref, batchf_ref, g_ref, w1_ref, w2_ref, w4_ref,
               a_ref, b_ref):
    gproj = jnp.dot(g_ref[...], w4_ref[...],
                    preferred_element_type=jnp.float32)
    iota = lax.broadcasted_iota(jnp.int32, (N_NODES, N_GRAPHS), 1)
    onehot = (batchf_ref[...] == iota).astype(jnp.float32)
    a_ref[...] = (
        jnp.dot(node_ref[...], w1_ref[...], preferred_element_type=jnp.float32)
        + jnp.dot(onehot, gproj, preferred_element_type=jnp.float32))
    b_ref[...] = jnp.dot(node_ref[...], w2_ref[...],
                         preferred_element_type=jnp.float32)


def _build_tables(node_feats, batch_f, global_feats, w1t, w2t, w4t):
    return pl.pallas_call(
        _proj_body,
        out_shape=[
            jax.ShapeDtypeStruct((N_NODES, HIDDEN), jnp.float32),
            jax.ShapeDtypeStruct((N_NODES, HIDDEN), jnp.float32),
        ],
    )(node_feats, batch_f, global_feats, w1t, w2t, w4t)


_NSLOT = 5
_HI16 = np.int32(-65536)


def _asf32(x):
    return lax.bitcast_convert_type(x, jnp.float32)


def _rne(x):
    bits = lax.bitcast_convert_type(x, jnp.int32)
    lsb = jnp.bitwise_and(lax.shift_right_logical(bits, 16), np.int32(1))
    return bits + np.int32(0x7FFF) + lsb


_NOUTER = _NCHUNK // _NSLOT


def _sc_body(a_hbm, b_hbm, src_hbm, dst_hbm, out_hbm,
             idx_s, idx_d, buf_a, buf_b,
             sem_a, sem_b, sem_st, sem_is, sem_id):
    wid = lax.axis_index("s") * 2 + lax.axis_index("c")
    wbase = wid * _PER_W

    def fire_idx(g, par):
        pltpu.async_copy(src_hbm.at[wid, g], idx_s.at[par], sem_is)
        pltpu.async_copy(dst_hbm.at[wid, g], idx_d.at[par], sem_id)

    def wait_idx(g, par):
        pltpu.make_async_copy(src_hbm.at[wid, g], idx_s.at[par],
                              sem_is).wait()
        pltpu.make_async_copy(dst_hbm.at[wid, g], idx_d.at[par],
                              sem_id).wait()

    def fire_gathers(par, b, slot):
        pltpu.async_copy(a_hbm.at[idx_s.at[par, b]], buf_a.at[slot],
                         sem_a.at[slot])
        pltpu.async_copy(b_hbm.at[idx_d.at[par, b]], buf_b.at[slot],
                         sem_b.at[slot])

    def wait_gathers(par, b, slot):
        pltpu.make_async_copy(a_hbm.at[idx_s.at[par, b]], buf_a.at[slot],
                              sem_a.at[slot]).wait()
        pltpu.make_async_copy(b_hbm.at[idx_d.at[par, b]], buf_b.at[slot],
                              sem_b.at[slot]).wait()

    def drain_store(slot):
        pltpu.make_async_copy(buf_b.at[slot, pl.ds(0, _CHUNK // 2)],
                              out_hbm.at[pl.ds(0, _CHUNK // 2)],
                              sem_st.at[slot]).wait()

    fire_idx(0, 0)
    wait_idx(0, 0)
    fire_gathers(0, 0, 0)

    def outer(g, carry):
        par = lax.rem(g, 2)
        npar = 1 - par
        for b in range(_NSLOT):
            j = g * _NSLOT + b
            nslot = (b + 1) % _NSLOT

            if b == 0:
                @pl.when(g < _NOUTER - 1)
                def _():
                    fire_idx(g + 1, npar)

            @pl.when(j + 1 < _NCHUNK)
            def _():
                @pl.when(j + 1 >= _NSLOT)
                def _():
                    drain_store(nslot)
                if b == _NSLOT - 1:
                    wait_idx(g + 1, npar)
                    fire_gathers(npar, 0, nslot)
                else:
                    fire_gathers(par, b + 1, nslot)

            wait_gathers(par, b, b)

            def rowpair(p, carry2):
                r0 = p * 2
                r1 = r0 + 1
                for c in range(HIDDEN // 16):
                    sl = pl.ds(c * 16, 16)
                    lo = buf_a[b, r0, sl] + buf_b[b, r0, sl]
                    hi = buf_a[b, r1, sl] + buf_b[b, r1, sl]
                    buf_b[b, p, sl] = _asf32(jnp.bitwise_or(
                        lax.shift_right_logical(_rne(lo), 16),
                        jnp.bitwise_and(_rne(hi), _HI16)))
                return carry2

            lax.fori_loop(0, _CHUNK // 2, rowpair, 0)
            obase = pl.multiple_of((wbase + j * _CHUNK) // 2, 8)
            pltpu.async_copy(
                buf_b.at[b, pl.ds(0, _CHUNK // 2)],
                out_hbm.at[pl.ds(obase, _CHUNK // 2)],
                sem_st.at[b])
        return carry

    lax.fori_loop(0, _NOUTER, outer, 0)
    for s in range(_NSLOT):
        drain_store(s)


def _gather_add(a_tbl, b_tbl, src, dst):
    mesh = plsc.VectorSubcoreMesh(core_axis_name="c", subcore_axis_name="s")
    fn = functools.partial(
        pl.kernel,
        out_type=jax.ShapeDtypeStruct((N_EDGES // 2, HIDDEN), jnp.float32),
        mesh=mesh,
        scratch_types=[
            pltpu.VMEM((2, _NSLOT, _CHUNK), jnp.int32),
            pltpu.VMEM((2, _NSLOT, _CHUNK), jnp.int32),
            pltpu.VMEM((_NSLOT, _CHUNK, HIDDEN), jnp.float32),
            pltpu.VMEM((_NSLOT, _CHUNK, HIDDEN), jnp.float32),
            pltpu.SemaphoreType.DMA((_NSLOT,)),
            pltpu.SemaphoreType.DMA((_NSLOT,)),
            pltpu.SemaphoreType.DMA((_NSLOT,)),
            pltpu.SemaphoreType.DMA,
            pltpu.SemaphoreType.DMA,
        ],
    )(_sc_body)
    src4 = src.reshape(_NW, _NOUTER, _NSLOT, _CHUNK)
    dst4 = dst.reshape(_NW, _NOUTER, _NSLOT, _CHUNK)
    return fn(a_tbl, b_tbl, src4, dst4)


_BLK = 2000


def _final_body(s_ref, ef_ref, w3_ref, o_ref):
    w = s_ref[...]
    s = pltpu.bitcast(w, jnp.bfloat16).astype(jnp.float32)
    e = jnp.dot(ef_ref[...], w3_ref[...], preferred_element_type=jnp.float32)
    h = s + e
    o_ref[...] = (jnp.maximum(h, 0.0)
                  + jnp.log1p(jnp.exp(-jnp.abs(h))) - _LOG2)


def _finalize(s, edge_feats, w3t):
    grid = (N_EDGES // _BLK,)
    return pl.pallas_call(
        _final_body,
        grid=grid,
        in_specs=[
            pl.BlockSpec((_BLK // 2, HIDDEN), lambda i: (i, 0)),
            pl.BlockSpec((_BLK, D_EDGE), lambda i: (i, 0)),
            pl.BlockSpec((D_EDGE, HIDDEN), lambda i: (0, 0)),
        ],
        out_specs=pl.BlockSpec((_BLK, HIDDEN), lambda i: (i, 0)),
        out_shape=jax.ShapeDtypeStruct((N_EDGES, HIDDEN), jnp.float32),
    )(s, edge_feats, w3t)


def kernel(node_feats, edge_feats, global_feats, edge_index, batch, W):
    wt = W.T
    w1t = wt[:D_NODE]
    w2t = wt[D_NODE:2 * D_NODE]
    w3t = wt[2 * D_NODE:2 * D_NODE + D_EDGE]
    w4t = wt[2 * D_NODE + D_EDGE:]
    batch_f = batch.astype(jnp.int32)[:, None]
    src = edge_index[0].astype(jnp.int32)
    dst = edge_index[1].astype(jnp.int32)

    a_tbl, b_tbl = _build_tables(node_feats, batch_f, global_feats,
                                 w1t, w2t, w4t)
    s = _gather_add(a_tbl, b_tbl, src, dst)
    return _finalize(s, edge_feats, w3t)


# speedup vs baseline: 1.1059x; 1.1059x over previous
import functools

import jax
import jax.numpy as jnp
import numpy as np
from jax import lax
from jax.experimental import pallas as pl
from jax.experimental.pallas import tpu as pltpu
from jax.experimental.pallas import tpu_sc as plsc

N_NODES = 10000
N_EDGES = 320000
D_NODE = 128
D_EDGE = 16
D_GLOBAL = 16
N_GRAPHS = 64
HIDDEN = 128

_NW = 32
_PER_W = N_EDGES // _NW
_CHUNK = 80
_NCHUNK = _PER_W // _CHUNK

_LOG2 = 0.6931471805599453


def _proj_body(node_ref, batchf_ref, g_ref, w1_ref, w2_ref, w4_ref,
               a_ref, b_ref):
    gproj = jnp.dot(g_ref[...], w4_ref[...],
                    preferred_element_type=jnp.float32)
    iota = lax.broadcasted_iota(jnp.int32, (N_NODES, N_GRAPHS), 1)
    onehot = (batchf_ref[...] == iota).astype(jnp.float32)
    a_ref[...] = (
        jnp.dot(node_ref[...], w1_ref[...], preferred_element_type=jnp.float32)
        + jnp.dot(onehot, gproj, preferred_element_type=jnp.float32))
    b_ref[...] = jnp.dot(node_ref[...], w2_ref[...],
                         preferred_element_type=jnp.float32)


def _build_tables(node_feats, batch_f, global_feats, w1t, w2t, w4t):
    return pl.pallas_call(
        _proj_body,
        out_shape=[
            jax.ShapeDtypeStruct((N_NODES, HIDDEN), jnp.float32),
            jax.ShapeDtypeStruct((N_NODES, HIDDEN), jnp.float32),
        ],
    )(node_feats, batch_f, global_feats, w1t, w2t, w4t)


_NSLOT = 5
_HI16 = np.int32(-65536)


def _asf32(x):
    return lax.bitcast_convert_type(x, jnp.float32)


def _rne(x):
    return lax.bitcast_convert_type(x, jnp.int32) + np.int32(0x8000)


_NOUTER = _NCHUNK // _NSLOT


def _sc_body(a_hbm, b_hbm, src_hbm, dst_hbm, out_hbm,
             idx_s, idx_d, buf_a, buf_b,
             sem_a, sem_b, sem_st, sem_is, sem_id):
    wid = lax.axis_index("s") * 2 + lax.axis_index("c")
    wbase = wid * _PER_W

    def fire_idx(g, par):
        pltpu.async_copy(src_hbm.at[wid, g], idx_s.at[par], sem_is)
        pltpu.async_copy(dst_hbm.at[wid, g], idx_d.at[par], sem_id)

    def wait_idx(g, par):
        pltpu.make_async_copy(src_hbm.at[wid, g], idx_s.at[par],
                              sem_is).wait()
        pltpu.make_async_copy(dst_hbm.at[wid, g], idx_d.at[par],
                              sem_id).wait()

    def fire_gathers(par, b, slot):
        pltpu.async_copy(a_hbm.at[idx_s.at[par, b]], buf_a.at[slot],
                         sem_a.at[slot])
        pltpu.async_copy(b_hbm.at[idx_d.at[par, b]], buf_b.at[slot],
                         sem_b.at[slot])

    def wait_gathers(par, b, slot):
        pltpu.make_async_copy(a_hbm.at[idx_s.at[par, b]], buf_a.at[slot],
                              sem_a.at[slot]).wait()
        pltpu.make_async_copy(b_hbm.at[idx_d.at[par, b]], buf_b.at[slot],
                              sem_b.at[slot]).wait()

    def drain_store(slot):
        pltpu.make_async_copy(buf_b.at[slot, pl.ds(0, _CHUNK // 2)],
                              out_hbm.at[pl.ds(0, _CHUNK // 2)],
                              sem_st.at[slot]).wait()

    fire_idx(0, 0)
    wait_idx(0, 0)
    fire_gathers(0, 0, 0)

    def outer(g, carry):
        par = lax.rem(g, 2)
        npar = 1 - par
        for b in range(_NSLOT):
            j = g * _NSLOT + b
            nslot = (b + 1) % _NSLOT

            if b == 0:
                @pl.when(g < _NOUTER - 1)
                def _():
                    fire_idx(g + 1, npar)

            @pl.when(j + 1 < _NCHUNK)
            def _():
                @pl.when(j + 1 >= _NSLOT)
                def _():
                    drain_store(nslot)
                if b == _NSLOT - 1:
                    wait_idx(g + 1, npar)
                    fire_gathers(npar, 0, nslot)
                else:
                    fire_gathers(par, b + 1, nslot)

            wait_gathers(par, b, b)

            def rowquad(q, carry2):
                for u in range(2):
                    p = q * 2 + u
                    r0 = p * 2
                    r1 = r0 + 1
                    for c in range(HIDDEN // 16):
                        sl = pl.ds(c * 16, 16)
                        lo = buf_a[b, r0, sl] + buf_b[b, r0, sl]
                        hi = buf_a[b, r1, sl] + buf_b[b, r1, sl]
                        buf_b[b, p, sl] = _asf32(jnp.bitwise_or(
                            lax.shift_right_logical(_rne(lo), 16),
                            jnp.bitwise_and(_rne(hi), _HI16)))
                return carry2

            lax.fori_loop(0, _CHUNK // 4, rowquad, 0)
            obase = pl.multiple_of((wbase + j * _CHUNK) // 2, 8)
            pltpu.async_copy(
                buf_b.at[b, pl.ds(0, _CHUNK // 2)],
                out_hbm.at[pl.ds(obase, _CHUNK // 2)],
                sem_st.at[b])
        return carry

    lax.fori_loop(0, _NOUTER, outer, 0)
    for s in range(_NSLOT):
        drain_store(s)


def _gather_add(a_tbl, b_tbl, src, dst):
    mesh = plsc.VectorSubcoreMesh(core_axis_name="c", subcore_axis_name="s")
    fn = functools.partial(
        pl.kernel,
        out_type=jax.ShapeDtypeStruct((N_EDGES // 2, HIDDEN), jnp.float32),
        mesh=mesh,
        scratch_types=[
            pltpu.VMEM((2, _NSLOT, _CHUNK), jnp.int32),
            pltpu.VMEM((2, _NSLOT, _CHUNK), jnp.int32),
            pltpu.VMEM((_NSLOT, _CHUNK, HIDDEN), jnp.float32),
            pltpu.VMEM((_NSLOT, _CHUNK, HIDDEN), jnp.float32),
            pltpu.SemaphoreType.DMA((_NSLOT,)),
            pltpu.SemaphoreType.DMA((_NSLOT,)),
            pltpu.SemaphoreType.DMA((_NSLOT,)),
            pltpu.SemaphoreType.DMA,
            pltpu.SemaphoreType.DMA,
        ],
    )(_sc_body)
    src4 = src.reshape(_NW, _NOUTER, _NSLOT, _CHUNK)
    dst4 = dst.reshape(_NW, _NOUTER, _NSLOT, _CHUNK)
    return fn(a_tbl, b_tbl, src4, dst4)


_BLK = 2000


def _final_body(s_ref, ef_ref, w3_ref, o_ref):
    w = s_ref[...]
    s = pltpu.bitcast(w, jnp.bfloat16).astype(jnp.float32)
    e = jnp.dot(ef_ref[...], w3_ref[...], preferred_element_type=jnp.float32)
    h = s + e
    o_ref[...] = (jnp.maximum(h, 0.0)
                  + jnp.log1p(jnp.exp(-jnp.abs(h))) - _LOG2)


def _finalize(s, edge_feats, w3t):
    grid = (N_EDGES // _BLK,)
    return pl.pallas_call(
        _final_body,
        grid=grid,
        in_specs=[
            pl.BlockSpec((_BLK // 2, HIDDEN), lambda i: (i, 0)),
            pl.BlockSpec((_BLK, D_EDGE), lambda i: (i, 0)),
            pl.BlockSpec((D_EDGE, HIDDEN), lambda i: (0, 0)),
        ],
        out_specs=pl.BlockSpec((_BLK, HIDDEN), lambda i: (i, 0)),
        out_shape=jax.ShapeDtypeStruct((N_EDGES, HIDDEN), jnp.float32),
    )(s, edge_feats, w3t)


def kernel(node_feats, edge_feats, global_feats, edge_index, batch, W):
    wt = W.T
    w1t = wt[:D_NODE]
    w2t = wt[D_NODE:2 * D_NODE]
    w3t = wt[2 * D_NODE:2 * D_NODE + D_EDGE]
    w4t = wt[2 * D_NODE + D_EDGE:]
    batch_f = batch.astype(jnp.int32)[:, None]
    src = edge_index[0].astype(jnp.int32)
    dst = edge_index[1].astype(jnp.int32)

    a_tbl, b_tbl = _build_tables(node_feats, batch_f, global_feats,
                                 w1t, w2t, w4t)
    s = _gather_add(a_tbl, b_tbl, src, dst)
    return _finalize(s, edge_feats, w3t)


# speedup vs baseline: 1.3639x; 1.2333x over previous
import functools

import jax
import jax.numpy as jnp
import numpy as np
from jax import lax
from jax.experimental import pallas as pl
from jax.experimental.pallas import tpu as pltpu
from jax.experimental.pallas import tpu_sc as plsc

N_NODES = 10000
N_EDGES = 320000
D_NODE = 128
D_EDGE = 16
D_GLOBAL = 16
N_GRAPHS = 64
HIDDEN = 128

_NW = 32
_CHUNK = 80
_NSLOT = 5
_NOUTER = 5
_NCHUNK = _NSLOT * _NOUTER
_PER_W = _NCHUNK * _CHUNK
_E_SLICE = _PER_W * _NW
_NSLICE = N_EDGES // _E_SLICE

_LOG2 = 0.6931471805599453


def _proj_body(node_ref, batchf_ref, g_ref, w1_ref, w2_ref, w4_ref,
               a_ref, b_ref):
    gproj = jnp.dot(g_ref[...], w4_ref[...],
                    preferred_element_type=jnp.float32)
    iota = lax.broadcasted_iota(jnp.int32, (N_NODES, N_GRAPHS), 1)
    onehot = (batchf_ref[...] == iota).astype(jnp.float32)
    a_ref[...] = (
        jnp.dot(node_ref[...], w1_ref[...], preferred_element_type=jnp.float32)
        + jnp.dot(onehot, gproj, preferred_element_type=jnp.float32))
    b_ref[...] = jnp.dot(node_ref[...], w2_ref[...],
                         preferred_element_type=jnp.float32)


def _build_tables(node_feats, batch_f, global_feats, w1t, w2t, w4t):
    return pl.pallas_call(
        _proj_body,
        out_shape=[
            jax.ShapeDtypeStruct((N_NODES, HIDDEN), jnp.float32),
            jax.ShapeDtypeStruct((N_NODES, HIDDEN), jnp.float32),
        ],
    )(node_feats, batch_f, global_feats, w1t, w2t, w4t)


def _sc_body(a_hbm, b_hbm, src_hbm, dst_hbm, out_hbm,
             idx_s, idx_d, buf_a, buf_b,
             sem_a, sem_b, sem_st, sem_is, sem_id):
    wid = lax.axis_index("s") * 2 + lax.axis_index("c")
    wbase = wid * _PER_W

    def fire_idx(g, par):
        pltpu.async_copy(src_hbm.at[wid, g], idx_s.at[par], sem_is)
        pltpu.async_copy(dst_hbm.at[wid, g], idx_d.at[par], sem_id)

    def wait_idx(g, par):
        pltpu.make_async_copy(src_hbm.at[wid, g], idx_s.at[par],
                              sem_is).wait()
        pltpu.make_async_copy(dst_hbm.at[wid, g], idx_d.at[par],
                              sem_id).wait()

    def fire_gathers(par, b, slot):
        pltpu.async_copy(a_hbm.at[idx_s.at[par, b]], buf_a.at[slot],
                         sem_a.at[slot])
        pltpu.async_copy(b_hbm.at[idx_d.at[par, b]], buf_b.at[slot],
                         sem_b.at[slot])

    def wait_gathers(par, b, slot):
        pltpu.make_async_copy(a_hbm.at[idx_s.at[par, b]], buf_a.at[slot],
                              sem_a.at[slot]).wait()
        pltpu.make_async_copy(b_hbm.at[idx_d.at[par, b]], buf_b.at[slot],
                              sem_b.at[slot]).wait()

    def drain_store(slot):
        pltpu.make_async_copy(buf_a.at[slot], out_hbm.at[pl.ds(0, _CHUNK)],
                              sem_st.at[slot]).wait()

    fire_idx(0, 0)
    wait_idx(0, 0)
    fire_gathers(0, 0, 0)

    def outer(g, carry):
        par = lax.rem(g, 2)
        npar = 1 - par
        for b in range(_NSLOT):
            j = g * _NSLOT + b
            nslot = (b + 1) % _NSLOT

            if b == 0:
                @pl.when(g < _NOUTER - 1)
                def _():
                    fire_idx(g + 1, npar)

            @pl.when(j + 1 < _NCHUNK)
            def _():
                @pl.when(j + 1 >= _NSLOT)
                def _():
                    drain_store(nslot)
                if b == _NSLOT - 1:
                    wait_idx(g + 1, npar)
                    fire_gathers(npar, 0, nslot)
                else:
                    fire_gathers(par, b + 1, nslot)

            wait_gathers(par, b, b)

            def row(r, carry2):
                r0 = pl.multiple_of(r * 2, 2)
                for u in range(2):
                    for c in range(HIDDEN // 16):
                        sl = pl.ds(c * 16, 16)
                        plsc.addupdate(buf_a.at[b, r0 + u, sl],
                                       buf_b[b, r0 + u, sl])
                return carry2

            lax.fori_loop(0, _CHUNK // 2, row, 0)
            pltpu.async_copy(buf_a.at[b],
                             out_hbm.at[pl.ds(wbase + j * _CHUNK, _CHUNK)],
                             sem_st.at[b])
        return carry

    lax.fori_loop(0, _NOUTER, outer, 0)
    for s in range(_NSLOT):
        drain_store(s)


def _gather_add(a_tbl, b_tbl, src4, dst4):
    mesh = plsc.VectorSubcoreMesh(core_axis_name="c", subcore_axis_name="s")
    fn = functools.partial(
        pl.kernel,
        out_type=jax.ShapeDtypeStruct((_E_SLICE, HIDDEN), jnp.float32),
        mesh=mesh,
        scratch_types=[
            pltpu.VMEM((2, _NSLOT, _CHUNK), jnp.int32),
            pltpu.VMEM((2, _NSLOT, _CHUNK), jnp.int32),
            pltpu.VMEM((_NSLOT, _CHUNK, HIDDEN), jnp.float32),
            pltpu.VMEM((_NSLOT, _CHUNK, HIDDEN), jnp.float32),
            pltpu.SemaphoreType.DMA((_NSLOT,)),
            pltpu.SemaphoreType.DMA((_NSLOT,)),
            pltpu.SemaphoreType.DMA((_NSLOT,)),
            pltpu.SemaphoreType.DMA,
            pltpu.SemaphoreType.DMA,
        ],
    )(_sc_body)
    return fn(a_tbl, b_tbl, src4, dst4)


_BLK = 2000
_BLK_PER_SLICE = _E_SLICE // _BLK


def _final_body(prev_ref, s_ref, ef_ref, w3_ref, o_ref):
    del prev_ref
    e = jnp.dot(ef_ref[...], w3_ref[...], preferred_element_type=jnp.float32)
    h = s_ref[...] + e
    o_ref[...] = (jnp.maximum(h, 0.0)
                  + jnp.log1p(jnp.exp(-jnp.abs(h))) - _LOG2)


def _final_body0(s_ref, ef_ref, w3_ref, o_ref):
    _final_body(None, s_ref, ef_ref, w3_ref, o_ref)


def _finalize_slice(prev_out, s_k, edge_feats, w3t, k):
    blk0 = k * _BLK_PER_SLICE
    common = dict(
        grid=(_BLK_PER_SLICE,),
        out_specs=pl.BlockSpec((_BLK, HIDDEN), lambda i, b0=blk0: (b0 + i, 0)),
        out_shape=jax.ShapeDtypeStruct((N_EDGES, HIDDEN), jnp.float32),
    )
    data_specs = [
        pl.BlockSpec((_BLK, HIDDEN), lambda i: (i, 0)),
        pl.BlockSpec((_BLK, D_EDGE), lambda i, b0=blk0: (b0 + i, 0)),
        pl.BlockSpec((D_EDGE, HIDDEN), lambda i: (0, 0)),
    ]
    if prev_out is None:
        return pl.pallas_call(
            _final_body0, in_specs=data_specs, **common,
        )(s_k, edge_feats, w3t)
    return pl.pallas_call(
        _final_body,
        in_specs=[pl.BlockSpec((8, HIDDEN), lambda i: (0, 0))] + data_specs,
        input_output_aliases={0: 0},
        **common,
    )(prev_out, s_k, edge_feats, w3t)


def kernel(node_feats, edge_feats, global_feats, edge_index, batch, W):
    wt = W.T
    w1t = wt[:D_NODE]
    w2t = wt[D_NODE:2 * D_NODE]
    w3t = wt[2 * D_NODE:2 * D_NODE + D_EDGE]
    w4t = wt[2 * D_NODE + D_EDGE:]
    batch_f = batch.astype(jnp.int32)[:, None]
    src = edge_index[0].astype(jnp.int32)
    dst = edge_index[1].astype(jnp.int32)

    a_tbl, b_tbl = _build_tables(node_feats, batch_f, global_feats,
                                 w1t, w2t, w4t)

    src5 = src.reshape(_NSLICE, _NW, _NOUTER, _NSLOT, _CHUNK)
    dst5 = dst.reshape(_NSLICE, _NW, _NOUTER, _NSLOT, _CHUNK)
    s_slices = [_gather_add(a_tbl, b_tbl, src5[k], dst5[k])
                for k in range(_NSLICE)]

    out = None
    for k in range(_NSLICE):
        out = _finalize_slice(out, s_slices[k], edge_feats, w3t, k)
    return out


# speedup vs baseline: 1.4328x; 1.0505x over previous
import functools

import jax
import jax.numpy as jnp
import numpy as np
from jax import lax
from jax.experimental import pallas as pl
from jax.experimental.pallas import tpu as pltpu
from jax.experimental.pallas import tpu_sc as plsc

N_NODES = 10000
N_EDGES = 320000
D_NODE = 128
D_EDGE = 16
D_GLOBAL = 16
N_GRAPHS = 64
HIDDEN = 128

_NW = 32
_CHUNK = 80
_NSLOT = 5
_NOUTER = 5
_NCHUNK = _NSLOT * _NOUTER
_PER_W = _NCHUNK * _CHUNK
_E_SLICE = _PER_W * _NW
_NSLICE = N_EDGES // _E_SLICE

_LOG2 = 0.6931471805599453
_HI16 = np.int32(-65536)


def _asf32(x):
    return lax.bitcast_convert_type(x, jnp.float32)


def _asi32(x):
    return lax.bitcast_convert_type(x, jnp.int32)


def _rne(x):
    return _asi32(x) + np.int32(0x8000)


def _proj_body(node_ref, batchf_ref, g_ref, w1_ref, w2_ref, w4_ref,
               a_ref, b_ref):
    gproj = jnp.dot(g_ref[...], w4_ref[...],
                    preferred_element_type=jnp.float32)
    iota = lax.broadcasted_iota(jnp.int32, (N_NODES, N_GRAPHS), 1)
    onehot = (batchf_ref[...] == iota).astype(jnp.float32)
    a_ref[...] = (
        jnp.dot(node_ref[...], w1_ref[...], preferred_element_type=jnp.float32)
        + jnp.dot(onehot, gproj, preferred_element_type=jnp.float32))
    b_ref[...] = jnp.dot(node_ref[...], w2_ref[...],
                         preferred_element_type=jnp.float32)


def _build_tables(node_feats, batch_f, global_feats, w1t, w2t, w4t):
    return pl.pallas_call(
        _proj_body,
        out_shape=[
            jax.ShapeDtypeStruct((N_NODES, HIDDEN), jnp.float32),
            jax.ShapeDtypeStruct((N_NODES, HIDDEN), jnp.float32),
        ],
    )(node_feats, batch_f, global_feats, w1t, w2t, w4t)


def _sc_body(a_hbm, b_hbm, src_hbm, dst_hbm, out_hbm,
             idx_s, idx_d, buf_a, buf_b,
             sem_a, sem_b, sem_st, sem_is, sem_id):
    wid = lax.axis_index("s") * 2 + lax.axis_index("c")
    wbase = wid * _PER_W

    def fire_idx(g, par):
        pltpu.async_copy(src_hbm.at[wid, g], idx_s.at[par], sem_is)
        pltpu.async_copy(dst_hbm.at[wid, g], idx_d.at[par], sem_id)

    def wait_idx(g, par):
        pltpu.make_async_copy(src_hbm.at[wid, g], idx_s.at[par],
                              sem_is).wait()
        pltpu.make_async_copy(dst_hbm.at[wid, g], idx_d.at[par],
                              sem_id).wait()

    def fire_gathers(par, b, slot):
        pltpu.async_copy(a_hbm.at[idx_s.at[par, b]], buf_a.at[slot],
                         sem_a.at[slot])
        pltpu.async_copy(b_hbm.at[idx_d.at[par, b]], buf_b.at[slot],
                         sem_b.at[slot])

    def wait_gathers(par, b, slot):
        pltpu.make_async_copy(a_hbm.at[idx_s.at[par, b]], buf_a.at[slot],
                              sem_a.at[slot]).wait()
        pltpu.make_async_copy(b_hbm.at[idx_d.at[par, b]], buf_b.at[slot],
                              sem_b.at[slot]).wait()

    def drain_store(slot):
        pltpu.make_async_copy(buf_b.at[slot, pl.ds(0, _CHUNK // 2)],
                              out_hbm.at[pl.ds(0, _CHUNK // 2)],
                              sem_st.at[slot]).wait()

    fire_idx(0, 0)
    wait_idx(0, 0)
    fire_gathers(0, 0, 0)

    def outer(g, carry):
        par = lax.rem(g, 2)
        npar = 1 - par
        for b in range(_NSLOT):
            j = g * _NSLOT + b
            nslot = (b + 1) % _NSLOT

            if b == 0:
                @pl.when(g < _NOUTER - 1)
                def _():
                    fire_idx(g + 1, npar)

            @pl.when(j + 1 < _NCHUNK)
            def _():
                @pl.when(j + 1 >= _NSLOT)
                def _():
                    drain_store(nslot)
                if b == _NSLOT - 1:
                    wait_idx(g + 1, npar)
                    fire_gathers(npar, 0, nslot)
                else:
                    fire_gathers(par, b + 1, nslot)

            wait_gathers(par, b, b)

            def rowquad(q, carry2):
                for u in range(2):
                    p = q * 2 + u
                    r0 = p * 2
                    r1 = r0 + 1
                    for c in range(HIDDEN // 16):
                        sl = pl.ds(c * 16, 16)
                        lo = buf_a[b, r0, sl] + buf_b[b, r0, sl]
                        hi = buf_a[b, r1, sl] + buf_b[b, r1, sl]
                        buf_b[b, p, sl] = _asf32(jnp.bitwise_or(
                            lax.shift_right_logical(_rne(lo), 16),
                            jnp.bitwise_and(_rne(hi), _HI16)))
                return carry2

            lax.fori_loop(0, _CHUNK // 4, rowquad, 0)
            obase = pl.multiple_of((wbase + j * _CHUNK) // 2, 8)
            pltpu.async_copy(buf_b.at[b, pl.ds(0, _CHUNK // 2)],
                             out_hbm.at[pl.ds(obase, _CHUNK // 2)],
                             sem_st.at[b])
        return carry

    lax.fori_loop(0, _NOUTER, outer, 0)
    for s in range(_NSLOT):
        drain_store(s)


def _gather_add(a_tbl, b_tbl, src4, dst4):
    mesh = plsc.VectorSubcoreMesh(core_axis_name="c", subcore_axis_name="s")
    fn = functools.partial(
        pl.kernel,
        out_type=jax.ShapeDtypeStruct((_E_SLICE // 2, HIDDEN), jnp.float32),
        mesh=mesh,
        scratch_types=[
            pltpu.VMEM((2, _NSLOT, _CHUNK), jnp.int32),
            pltpu.VMEM((2, _NSLOT, _CHUNK), jnp.int32),
            pltpu.VMEM((_NSLOT, _CHUNK, HIDDEN), jnp.float32),
            pltpu.VMEM((_NSLOT, _CHUNK, HIDDEN), jnp.float32),
            pltpu.SemaphoreType.DMA((_NSLOT,)),
            pltpu.SemaphoreType.DMA((_NSLOT,)),
            pltpu.SemaphoreType.DMA((_NSLOT,)),
            pltpu.SemaphoreType.DMA,
            pltpu.SemaphoreType.DMA,
        ],
    )(_sc_body)
    return fn(a_tbl, b_tbl, src4, dst4)


_BLK = 2000
_BLK_PER_SLICE = _E_SLICE // _BLK


def _final_body(prev_ref, s_ref, ef_ref, w3_ref, o_ref):
    del prev_ref
    s = pltpu.bitcast(s_ref[...], jnp.bfloat16).astype(jnp.float32)
    e = jnp.dot(ef_ref[...], w3_ref[...], preferred_element_type=jnp.float32)
    h = s + e
    o_ref[...] = (jnp.maximum(h, 0.0)
                  + jnp.log1p(jnp.exp(-jnp.abs(h))) - _LOG2)


def _final_body0(s_ref, ef_ref, w3_ref, o_ref):
    _final_body(None, s_ref, ef_ref, w3_ref, o_ref)


def _finalize_slice(prev_out, s_k, edge_feats, w3t, k):
    blk0 = k * _BLK_PER_SLICE
    common = dict(
        grid=(_BLK_PER_SLICE,),
        out_specs=pl.BlockSpec((_BLK, HIDDEN), lambda i, b0=blk0: (b0 + i, 0)),
        out_shape=jax.ShapeDtypeStruct((N_EDGES, HIDDEN), jnp.float32),
    )
    data_specs = [
        pl.BlockSpec((_BLK // 2, HIDDEN), lambda i: (i, 0)),
        pl.BlockSpec((_BLK, D_EDGE), lambda i, b0=blk0: (b0 + i, 0)),
        pl.BlockSpec((D_EDGE, HIDDEN), lambda i: (0, 0)),
    ]
    if prev_out is None:
        return pl.pallas_call(
            _final_body0, in_specs=data_specs, **common,
        )(s_k, edge_feats, w3t)
    return pl.pallas_call(
        _final_body,
        in_specs=[pl.BlockSpec((8, HIDDEN), lambda i: (0, 0))] + data_specs,
        input_output_aliases={0: 0},
        **common,
    )(prev_out, s_k, edge_feats, w3t)


def kernel(node_feats, edge_feats, global_feats, edge_index, batch, W):
    wt = W.T
    w1t = wt[:D_NODE]
    w2t = wt[D_NODE:2 * D_NODE]
    w3t = wt[2 * D_NODE:2 * D_NODE + D_EDGE]
    w4t = wt[2 * D_NODE + D_EDGE:]
    batch_f = batch.astype(jnp.int32)[:, None]
    src = edge_index[0].astype(jnp.int32)
    dst = edge_index[1].astype(jnp.int32)

    a_tbl, b_tbl = _build_tables(node_feats, batch_f, global_feats,
                                 w1t, w2t, w4t)

    src5 = src.reshape(_NSLICE, _NW, _NOUTER, _NSLOT, _CHUNK)
    dst5 = dst.reshape(_NSLICE, _NW, _NOUTER, _NSLOT, _CHUNK)
    s_slices = [_gather_add(a_tbl, b_tbl, src5[k], dst5[k])
                for k in range(_NSLICE)]

    out = None
    for k in range(_NSLICE):
        out = _finalize_slice(out, s_slices[k], edge_feats, w3t, k)
    return out


# speedup vs baseline: 1.5122x; 1.0554x over previous
import functools

import jax
import jax.numpy as jnp
import numpy as np
from jax import lax
from jax.experimental import pallas as pl
from jax.experimental.pallas import tpu as pltpu
from jax.experimental.pallas import tpu_sc as plsc

N_NODES = 10000
N_EDGES = 320000
D_NODE = 128
D_EDGE = 16
D_GLOBAL = 16
N_GRAPHS = 64
HIDDEN = 128

_NW = 32
_CHUNK = 80
_NSLOT = 5
_NOUTER = 5
_NCHUNK = _NSLOT * _NOUTER
_PER_W = _NCHUNK * _CHUNK
_E_SLICE = _PER_W * _NW
_NSLICE = N_EDGES // _E_SLICE

_LOG2 = 0.6931471805599453
_HI16 = np.int32(-65536)


def _asf32(x):
    return lax.bitcast_convert_type(x, jnp.float32)


def _asi32(x):
    return lax.bitcast_convert_type(x, jnp.int32)


def _rne(x):
    return _asi32(x) + np.int32(0x8000)


def _proj_body(node_ref, batchf_ref, g_ref, w1_ref, w2_ref, w4_ref,
               a_ref, b_ref):
    gproj = jnp.dot(g_ref[...], w4_ref[...],
                    preferred_element_type=jnp.float32)
    iota = lax.broadcasted_iota(jnp.int32, (N_NODES, N_GRAPHS), 1)
    onehot = (batchf_ref[...] == iota).astype(jnp.float32)
    a_ref[...] = (
        jnp.dot(node_ref[...], w1_ref[...], preferred_element_type=jnp.float32)
        + jnp.dot(onehot, gproj, preferred_element_type=jnp.float32))
    b_ref[...] = jnp.dot(node_ref[...], w2_ref[...],
                         preferred_element_type=jnp.float32)


def _build_tables(node_feats, batch_f, global_feats, w1t, w2t, w4t):
    return pl.pallas_call(
        _proj_body,
        out_shape=[
            jax.ShapeDtypeStruct((N_NODES, HIDDEN), jnp.float32),
            jax.ShapeDtypeStruct((N_NODES, HIDDEN), jnp.float32),
        ],
    )(node_feats, batch_f, global_feats, w1t, w2t, w4t)


def _sc_body(a_hbm, b_hbm, src_hbm, dst_hbm, out_hbm,
             idx_s, idx_d, buf_a, buf_b,
             sem_a, sem_b, sem_st, sem_is, sem_id):
    wid = lax.axis_index("s") * 2 + lax.axis_index("c")
    wbase = wid * _PER_W

    def fire_idx(g, par):
        pltpu.async_copy(src_hbm.at[wid, g], idx_s.at[par], sem_is)
        pltpu.async_copy(dst_hbm.at[wid, g], idx_d.at[par], sem_id)

    def wait_idx(g, par):
        pltpu.make_async_copy(src_hbm.at[wid, g], idx_s.at[par],
                              sem_is).wait()
        pltpu.make_async_copy(dst_hbm.at[wid, g], idx_d.at[par],
                              sem_id).wait()

    def fire_gathers(par, b, slot):
        pltpu.async_copy(a_hbm.at[idx_s.at[par, b]], buf_a.at[slot],
                         sem_a.at[slot])
        pltpu.async_copy(b_hbm.at[idx_d.at[par, b]], buf_b.at[slot],
                         sem_b.at[slot])

    def wait_gathers(par, b, slot):
        pltpu.make_async_copy(a_hbm.at[idx_s.at[par, b]], buf_a.at[slot],
                              sem_a.at[slot]).wait()
        pltpu.make_async_copy(b_hbm.at[idx_d.at[par, b]], buf_b.at[slot],
                              sem_b.at[slot]).wait()

    def drain_store(slot):
        pltpu.make_async_copy(buf_b.at[slot, pl.ds(0, _CHUNK // 2)],
                              out_hbm.at[pl.ds(0, _CHUNK // 2)],
                              sem_st.at[slot]).wait()

    fire_idx(0, 0)
    wait_idx(0, 0)
    fire_gathers(0, 0, 0)

    def outer(g, carry):
        par = lax.rem(g, 2)
        npar = 1 - par
        for b in range(_NSLOT):
            j = g * _NSLOT + b
            nslot = (b + 1) % _NSLOT

            if b == 0:
                @pl.when(g < _NOUTER - 1)
                def _():
                    fire_idx(g + 1, npar)

            @pl.when(j + 1 < _NCHUNK)
            def _():
                @pl.when(j + 1 >= _NSLOT)
                def _():
                    drain_store(nslot)
                if b == _NSLOT - 1:
                    wait_idx(g + 1, npar)
                    fire_gathers(npar, 0, nslot)
                else:
                    fire_gathers(par, b + 1, nslot)

            wait_gathers(par, b, b)

            def rowquad(q, carry2):
                for u in range(2):
                    p = q * 2 + u
                    r0 = p * 2
                    r1 = r0 + 1
                    words = []
                    for c in range(HIDDEN // 16):
                        sl = pl.ds(c * 16, 16)
                        lo = buf_a[b, r0, sl] + buf_b[b, r0, sl]
                        hi = buf_a[b, r1, sl] + buf_b[b, r1, sl]
                        words.append(_asf32(jnp.bitwise_or(
                            lax.shift_right_logical(_rne(lo), 16),
                            jnp.bitwise_and(_rne(hi), _HI16))))
                    for c in range(HIDDEN // 16):
                        buf_b[b, p, pl.ds(c * 16, 16)] = words[c]
                return carry2

            lax.fori_loop(0, _CHUNK // 4, rowquad, 0)
            obase = pl.multiple_of((wbase + j * _CHUNK) // 2, 8)
            pltpu.async_copy(buf_b.at[b, pl.ds(0, _CHUNK // 2)],
                             out_hbm.at[pl.ds(obase, _CHUNK // 2)],
                             sem_st.at[b])
        return carry

    lax.fori_loop(0, _NOUTER, outer, 0)
    for s in range(_NSLOT):
        drain_store(s)


def _gather_add(a_tbl, b_tbl, src4, dst4):
    mesh = plsc.VectorSubcoreMesh(core_axis_name="c", subcore_axis_name="s")
    fn = functools.partial(
        pl.kernel,
        out_type=jax.ShapeDtypeStruct((_E_SLICE // 2, HIDDEN), jnp.float32),
        mesh=mesh,
        scratch_types=[
            pltpu.VMEM((2, _NSLOT, _CHUNK), jnp.int32),
            pltpu.VMEM((2, _NSLOT, _CHUNK), jnp.int32),
            pltpu.VMEM((_NSLOT, _CHUNK, HIDDEN), jnp.float32),
            pltpu.VMEM((_NSLOT, _CHUNK, HIDDEN), jnp.float32),
            pltpu.SemaphoreType.DMA((_NSLOT,)),
            pltpu.SemaphoreType.DMA((_NSLOT,)),
            pltpu.SemaphoreType.DMA((_NSLOT,)),
            pltpu.SemaphoreType.DMA,
            pltpu.SemaphoreType.DMA,
        ],
    )(_sc_body)
    return fn(a_tbl, b_tbl, src4, dst4)


_BLK = 2000
_BLK_PER_SLICE = _E_SLICE // _BLK


def _final_body(prev_ref, s_ref, ef_ref, w3_ref, o_ref):
    del prev_ref
    s = pltpu.bitcast(s_ref[...], jnp.bfloat16).astype(jnp.float32)
    e = jnp.dot(ef_ref[...], w3_ref[...], preferred_element_type=jnp.float32)
    h = s + e
    o_ref[...] = (jnp.maximum(h, 0.0)
                  + jnp.log1p(jnp.exp(-jnp.abs(h))) - _LOG2)


def _final_body0(s_ref, ef_ref, w3_ref, o_ref):
    _final_body(None, s_ref, ef_ref, w3_ref, o_ref)


def _finalize_slice(prev_out, s_k, edge_feats, w3t, k):
    blk0 = k * _BLK_PER_SLICE
    common = dict(
        grid=(_BLK_PER_SLICE,),
        out_specs=pl.BlockSpec((_BLK, HIDDEN), lambda i, b0=blk0: (b0 + i, 0)),
        out_shape=jax.ShapeDtypeStruct((N_EDGES, HIDDEN), jnp.float32),
    )
    data_specs = [
        pl.BlockSpec((_BLK // 2, HIDDEN), lambda i: (i, 0)),
        pl.BlockSpec((_BLK, D_EDGE), lambda i, b0=blk0: (b0 + i, 0)),
        pl.BlockSpec((D_EDGE, HIDDEN), lambda i: (0, 0)),
    ]
    if prev_out is None:
        return pl.pallas_call(
            _final_body0, in_specs=data_specs, **common,
        )(s_k, edge_feats, w3t)
    return pl.pallas_call(
        _final_body,
        in_specs=[pl.BlockSpec((8, HIDDEN), lambda i: (0, 0))] + data_specs,
        input_output_aliases={0: 0},
        **common,
    )(prev_out, s_k, edge_feats, w3t)


def kernel(node_feats, edge_feats, global_feats, edge_index, batch, W):
    wt = W.T
    w1t = wt[:D_NODE]
    w2t = wt[D_NODE:2 * D_NODE]
    w3t = wt[2 * D_NODE:2 * D_NODE + D_EDGE]
    w4t = wt[2 * D_NODE + D_EDGE:]
    batch_f = batch.astype(jnp.int32)[:, None]
    src = edge_index[0].astype(jnp.int32)
    dst = edge_index[1].astype(jnp.int32)

    a_tbl, b_tbl = _build_tables(node_feats, batch_f, global_feats,
                                 w1t, w2t, w4t)

    src5 = src.reshape(_NSLICE, _NW, _NOUTER, _NSLOT, _CHUNK)
    dst5 = dst.reshape(_NSLICE, _NW, _NOUTER, _NSLOT, _CHUNK)
    s_slices = [_gather_add(a_tbl, b_tbl, src5[k], dst5[k])
                for k in range(_NSLICE)]

    out = None
    for k in range(_NSLICE):
        out = _finalize_slice(out, s_slices[k], edge_feats, w3t, k)
    return out


# speedup vs baseline: 1.6449x; 1.0877x over previous
import functools

import jax
import jax.numpy as jnp
import numpy as np
from jax import lax
from jax.experimental import pallas as pl
from jax.experimental.pallas import tpu as pltpu
from jax.experimental.pallas import tpu_sc as plsc

N_NODES = 10000
N_EDGES = 320000
D_NODE = 128
D_EDGE = 16
D_GLOBAL = 16
N_GRAPHS = 64
HIDDEN = 128

_NW = 32
_CHUNK = 80
_NSLOT = 5
_NOUTER = 5
_NCHUNK = _NSLOT * _NOUTER
_PER_W = _NCHUNK * _CHUNK
_E_SLICE = _PER_W * _NW
_NSLICE = N_EDGES // _E_SLICE

_LOG2 = 0.6931471805599453
_HI16 = np.int32(-65536)


def _asf32(x):
    return lax.bitcast_convert_type(x, jnp.float32)


def _asi32(x):
    return lax.bitcast_convert_type(x, jnp.int32)


def _rne(x):
    return _asi32(x) + np.int32(0x8000)


def _proj_body(node_ref, batchf_ref, g_ref, w1_ref, w2_ref, w4_ref,
               a_ref, b_ref):
    gproj = jnp.dot(g_ref[...], w4_ref[...],
                    preferred_element_type=jnp.float32)
    iota = lax.broadcasted_iota(jnp.int32, (N_NODES, N_GRAPHS), 1)
    onehot = (batchf_ref[...] == iota).astype(jnp.float32)
    a_ref[...] = (
        jnp.dot(node_ref[...], w1_ref[...], preferred_element_type=jnp.float32)
        + jnp.dot(onehot, gproj, preferred_element_type=jnp.float32))
    b_ref[...] = jnp.dot(node_ref[...], w2_ref[...],
                         preferred_element_type=jnp.float32)


def _build_tables(node_feats, batch_f, global_feats, w1t, w2t, w4t):
    return pl.pallas_call(
        _proj_body,
        out_shape=[
            jax.ShapeDtypeStruct((N_NODES, HIDDEN), jnp.float32),
            jax.ShapeDtypeStruct((N_NODES, HIDDEN), jnp.float32),
        ],
    )(node_feats, batch_f, global_feats, w1t, w2t, w4t)


def _sc_body(a_hbm, b_hbm, src_hbm, dst_hbm, out_hbm,
             idx_s, idx_d, buf_a, buf_b,
             sem_a, sem_b, sem_st, sem_is, sem_id):
    wid = lax.axis_index("s") * 2 + lax.axis_index("c")
    wbase = wid * _PER_W

    def fire_idx(g, par):
        pltpu.async_copy(src_hbm.at[wid, g], idx_s.at[par], sem_is)
        pltpu.async_copy(dst_hbm.at[wid, g], idx_d.at[par], sem_id)

    def wait_idx(g, par):
        pltpu.make_async_copy(src_hbm.at[wid, g], idx_s.at[par],
                              sem_is).wait()
        pltpu.make_async_copy(dst_hbm.at[wid, g], idx_d.at[par],
                              sem_id).wait()

    def fire_gathers(par, b, slot):
        pltpu.async_copy(a_hbm.at[idx_s.at[par, b]], buf_a.at[slot],
                         sem_a.at[slot])
        pltpu.async_copy(b_hbm.at[idx_d.at[par, b]], buf_b.at[slot],
                         sem_b.at[slot])

    def wait_gathers(par, b, slot):
        pltpu.make_async_copy(a_hbm.at[idx_s.at[par, b]], buf_a.at[slot],
                              sem_a.at[slot]).wait()
        pltpu.make_async_copy(b_hbm.at[idx_d.at[par, b]], buf_b.at[slot],
                              sem_b.at[slot]).wait()

    def drain_store(slot):
        pltpu.make_async_copy(buf_b.at[slot, pl.ds(0, _CHUNK // 2)],
                              out_hbm.at[pl.ds(0, _CHUNK // 2)],
                              sem_st.at[slot]).wait()

    fire_idx(0, 0)
    wait_idx(0, 0)
    fire_gathers(0, 0, 0)

    def outer(g, carry):
        par = lax.rem(g, 2)
        npar = 1 - par
        for b in range(_NSLOT):
            j = g * _NSLOT + b
            nslot = (b + 1) % _NSLOT

            if b == 0:
                @pl.when(g < _NOUTER - 1)
                def _():
                    fire_idx(g + 1, npar)

            @pl.when(j + 1 < _NCHUNK)
            def _():
                @pl.when(j + 1 >= _NSLOT)
                def _():
                    drain_store(nslot)
                if b == _NSLOT - 1:
                    wait_idx(g + 1, npar)
                    fire_gathers(npar, 0, nslot)
                else:
                    fire_gathers(par, b + 1, nslot)

            wait_gathers(par, b, b)

            def rowquad(q, carry2):
                for u in range(2):
                    p = q * 2 + u
                    r0 = p * 2
                    r1 = r0 + 1
                    words = []
                    for c in range(HIDDEN // 16):
                        sl = pl.ds(c * 16, 16)
                        lo = buf_a[b, r0, sl] + buf_b[b, r0, sl]
                        hi = buf_a[b, r1, sl] + buf_b[b, r1, sl]
                        words.append(_asf32(jnp.bitwise_or(
                            lax.shift_right_logical(_rne(lo), 16),
                            jnp.bitwise_and(_rne(hi), _HI16))))
                    for c in range(HIDDEN // 16):
                        buf_b[b, p, pl.ds(c * 16, 16)] = words[c]
                return carry2

            lax.fori_loop(0, _CHUNK // 4, rowquad, 0)
            obase = pl.multiple_of((wbase + j * _CHUNK) // 2, 8)
            pltpu.async_copy(buf_b.at[b, pl.ds(0, _CHUNK // 2)],
                             out_hbm.at[pl.ds(obase, _CHUNK // 2)],
                             sem_st.at[b])
        return carry

    lax.fori_loop(0, _NOUTER, outer, 0)
    for s in range(_NSLOT):
        drain_store(s)


def _gather_add(a_tbl, b_tbl, src4, dst4):
    mesh = plsc.VectorSubcoreMesh(core_axis_name="c", subcore_axis_name="s")
    fn = functools.partial(
        pl.kernel,
        out_type=jax.ShapeDtypeStruct((_E_SLICE // 2, HIDDEN), jnp.float32),
        mesh=mesh,
        scratch_types=[
            pltpu.VMEM((2, _NSLOT, _CHUNK), jnp.int32),
            pltpu.VMEM((2, _NSLOT, _CHUNK), jnp.int32),
            pltpu.VMEM((_NSLOT, _CHUNK, HIDDEN), jnp.float32),
            pltpu.VMEM((_NSLOT, _CHUNK, HIDDEN), jnp.float32),
            pltpu.SemaphoreType.DMA((_NSLOT,)),
            pltpu.SemaphoreType.DMA((_NSLOT,)),
            pltpu.SemaphoreType.DMA((_NSLOT,)),
            pltpu.SemaphoreType.DMA,
            pltpu.SemaphoreType.DMA,
        ],
    )(_sc_body)
    return fn(a_tbl, b_tbl, src4, dst4)


_BLK = 4000
_BLK_PER_SLICE = _E_SLICE // _BLK


def _final_body(prev_ref, s_ref, ef_ref, w3_ref, o_ref):
    del prev_ref
    s = pltpu.bitcast(s_ref[...], jnp.bfloat16).astype(jnp.float32)
    e = jnp.dot(ef_ref[...], w3_ref[...], preferred_element_type=jnp.float32)
    h = s + e
    o_ref[...] = (jnp.maximum(h, 0.0)
                  + jnp.log1p(jnp.exp(-jnp.abs(h))) - _LOG2)


def _final_body0(s_ref, ef_ref, w3_ref, o_ref):
    _final_body(None, s_ref, ef_ref, w3_ref, o_ref)


def _finalize_slice(prev_out, s_k, edge_feats, w3t, k):
    blk0 = k * _BLK_PER_SLICE
    common = dict(
        grid=(_BLK_PER_SLICE,),
        out_specs=pl.BlockSpec((_BLK, HIDDEN), lambda i, b0=blk0: (b0 + i, 0)),
        out_shape=jax.ShapeDtypeStruct((N_EDGES, HIDDEN), jnp.float32),
    )
    data_specs = [
        pl.BlockSpec((_BLK // 2, HIDDEN), lambda i: (i, 0)),
        pl.BlockSpec((_BLK, D_EDGE), lambda i, b0=blk0: (b0 + i, 0)),
        pl.BlockSpec((D_EDGE, HIDDEN), lambda i: (0, 0)),
    ]
    if prev_out is None:
        return pl.pallas_call(
            _final_body0, in_specs=data_specs, **common,
        )(s_k, edge_feats, w3t)
    return pl.pallas_call(
        _final_body,
        in_specs=[pl.BlockSpec((8, HIDDEN), lambda i: (0, 0))] + data_specs,
        input_output_aliases={0: 0},
        **common,
    )(prev_out, s_k, edge_feats, w3t)


def kernel(node_feats, edge_feats, global_feats, edge_index, batch, W):
    wt = W.T
    w1t = wt[:D_NODE]
    w2t = wt[D_NODE:2 * D_NODE]
    w3t = wt[2 * D_NODE:2 * D_NODE + D_EDGE]
    w4t = wt[2 * D_NODE + D_EDGE:]
    batch_f = batch.astype(jnp.int32)[:, None]
    src = edge_index[0].astype(jnp.int32)
    dst = edge_index[1].astype(jnp.int32)

    a_tbl, b_tbl = _build_tables(node_feats, batch_f, global_feats,
                                 w1t, w2t, w4t)

    src5 = src.reshape(_NSLICE, _NW, _NOUTER, _NSLOT, _CHUNK)
    dst5 = dst.reshape(_NSLICE, _NW, _NOUTER, _NSLOT, _CHUNK)
    s_slices = [_gather_add(a_tbl, b_tbl, src5[k], dst5[k])
                for k in range(_NSLICE)]

    out = None
    for k in range(_NSLICE):
        out = _finalize_slice(out, s_slices[k], edge_feats, w3t, k)
    return out


# speedup vs baseline: 1.6877x; 1.0260x over previous
import functools

import jax
import jax.numpy as jnp
import numpy as np
from jax import lax
from jax.experimental import pallas as pl
from jax.experimental.pallas import tpu as pltpu
from jax.experimental.pallas import tpu_sc as plsc

N_NODES = 10000
N_EDGES = 320000
D_NODE = 128
D_EDGE = 16
D_GLOBAL = 16
N_GRAPHS = 64
HIDDEN = 128

_NW = 32
_CHUNK = 80
_NSLOT = 5
_NOUTER = 5
_NCHUNK = _NSLOT * _NOUTER
_PER_W = _NCHUNK * _CHUNK
_E_SLICE = _PER_W * _NW
_NSLICE = N_EDGES // _E_SLICE

_LOG2 = 0.6931471805599453
_HI16 = np.int32(-65536)


def _asf32(x):
    return lax.bitcast_convert_type(x, jnp.float32)


def _asi32(x):
    return lax.bitcast_convert_type(x, jnp.int32)


def _rne(x):
    return _asi32(x) + np.int32(0x8000)


def _proj_body(node_ref, batchf_ref, g_ref, w1_ref, w2_ref, w4_ref,
               a_ref, b_ref):
    gproj = jnp.dot(g_ref[...], w4_ref[...],
                    preferred_element_type=jnp.float32)
    iota = lax.broadcasted_iota(jnp.int32, (N_NODES, N_GRAPHS), 1)
    onehot = (batchf_ref[...] == iota).astype(jnp.float32)
    a_ref[...] = (
        jnp.dot(node_ref[...], w1_ref[...], preferred_element_type=jnp.float32)
        + jnp.dot(onehot, gproj, preferred_element_type=jnp.float32))
    b_ref[...] = jnp.dot(node_ref[...], w2_ref[...],
                         preferred_element_type=jnp.float32)


def _build_tables(node_feats, batch_f, global_feats, w1t, w2t, w4t):
    return pl.pallas_call(
        _proj_body,
        out_shape=[
            jax.ShapeDtypeStruct((N_NODES, HIDDEN), jnp.float32),
            jax.ShapeDtypeStruct((N_NODES, HIDDEN), jnp.float32),
        ],
    )(node_feats, batch_f, global_feats, w1t, w2t, w4t)


def _sc_body(a_hbm, b_hbm, src_hbm, dst_hbm, out_hbm,
             idx_s, idx_d, buf_a, buf_b,
             sem_a, sem_b, sem_st, sem_is, sem_id):
    wid = lax.axis_index("s") * 2 + lax.axis_index("c")
    wbase = wid * _PER_W

    def fire_idx(g, par):
        pltpu.async_copy(src_hbm.at[wid, g], idx_s.at[par], sem_is)
        pltpu.async_copy(dst_hbm.at[wid, g], idx_d.at[par], sem_id)

    def wait_idx(g, par):
        pltpu.make_async_copy(src_hbm.at[wid, g], idx_s.at[par],
                              sem_is).wait()
        pltpu.make_async_copy(dst_hbm.at[wid, g], idx_d.at[par],
                              sem_id).wait()

    def fire_gathers(par, b, slot):
        pltpu.async_copy(a_hbm.at[idx_s.at[par, b]], buf_a.at[slot],
                         sem_a.at[slot])
        pltpu.async_copy(b_hbm.at[idx_d.at[par, b]], buf_b.at[slot],
                         sem_b.at[slot])

    def wait_gathers(par, b, slot):
        pltpu.make_async_copy(a_hbm.at[idx_s.at[par, b]], buf_a.at[slot],
                              sem_a.at[slot]).wait()
        pltpu.make_async_copy(b_hbm.at[idx_d.at[par, b]], buf_b.at[slot],
                              sem_b.at[slot]).wait()

    def drain_store(slot):
        pltpu.make_async_copy(buf_b.at[slot, pl.ds(0, _CHUNK // 2)],
                              out_hbm.at[pl.ds(0, _CHUNK // 2)],
                              sem_st.at[slot]).wait()

    fire_idx(0, 0)
    wait_idx(0, 0)
    fire_gathers(0, 0, 0)

    def outer(g, carry):
        par = lax.rem(g, 2)
        npar = 1 - par
        for b in range(_NSLOT):
            j = g * _NSLOT + b
            nslot = (b + 1) % _NSLOT

            if b == 0:
                @pl.when(g < _NOUTER - 1)
                def _():
                    fire_idx(g + 1, npar)

            @pl.when(j + 1 < _NCHUNK)
            def _():
                @pl.when(j + 1 >= _NSLOT)
                def _():
                    drain_store(nslot)
                if b == _NSLOT - 1:
                    wait_idx(g + 1, npar)
                    fire_gathers(npar, 0, nslot)
                else:
                    fire_gathers(par, b + 1, nslot)

            wait_gathers(par, b, b)

            def rowquad(q, carry2):
                for u in range(2):
                    p = q * 2 + u
                    r0 = p * 2
                    r1 = r0 + 1
                    words = []
                    for c in range(HIDDEN // 16):
                        sl = pl.ds(c * 16, 16)
                        lo = buf_a[b, r0, sl] + buf_b[b, r0, sl]
                        hi = buf_a[b, r1, sl] + buf_b[b, r1, sl]
                        words.append(_asf32(jnp.bitwise_or(
                            lax.shift_right_logical(_rne(lo), 16),
                            jnp.bitwise_and(_rne(hi), _HI16))))
                    for c in range(HIDDEN // 16):
                        buf_b[b, p, pl.ds(c * 16, 16)] = words[c]
                return carry2

            lax.fori_loop(0, _CHUNK // 4, rowquad, 0)
            obase = pl.multiple_of((wbase + j * _CHUNK) // 2, 8)
            pltpu.async_copy(buf_b.at[b, pl.ds(0, _CHUNK // 2)],
                             out_hbm.at[pl.ds(obase, _CHUNK // 2)],
                             sem_st.at[b])
        return carry

    lax.fori_loop(0, _NOUTER, outer, 0)
    for s in range(_NSLOT):
        drain_store(s)


def _gather_add(a_tbl, b_tbl, src4, dst4):
    mesh = plsc.VectorSubcoreMesh(core_axis_name="c", subcore_axis_name="s")
    fn = functools.partial(
        pl.kernel,
        out_type=jax.ShapeDtypeStruct((_E_SLICE // 2, HIDDEN), jnp.float32),
        mesh=mesh,
        scratch_types=[
            pltpu.VMEM((2, _NSLOT, _CHUNK), jnp.int32),
            pltpu.VMEM((2, _NSLOT, _CHUNK), jnp.int32),
            pltpu.VMEM((_NSLOT, _CHUNK, HIDDEN), jnp.float32),
            pltpu.VMEM((_NSLOT, _CHUNK, HIDDEN), jnp.float32),
            pltpu.SemaphoreType.DMA((_NSLOT,)),
            pltpu.SemaphoreType.DMA((_NSLOT,)),
            pltpu.SemaphoreType.DMA((_NSLOT,)),
            pltpu.SemaphoreType.DMA,
            pltpu.SemaphoreType.DMA,
        ],
    )(_sc_body)
    return fn(a_tbl, b_tbl, src4, dst4)


_BLK = 8000
_BLK_PER_SLICE = _E_SLICE // _BLK


def _final_body(prev_ref, s_ref, ef_ref, w3_ref, o_ref):
    del prev_ref
    s = pltpu.bitcast(s_ref[...], jnp.bfloat16).astype(jnp.float32)
    e = jnp.dot(ef_ref[...], w3_ref[...], preferred_element_type=jnp.float32)
    h = s + e
    o_ref[...] = (jnp.maximum(h, 0.0)
                  + jnp.log1p(jnp.exp(-jnp.abs(h))) - _LOG2)


def _final_body0(s_ref, ef_ref, w3_ref, o_ref):
    _final_body(None, s_ref, ef_ref, w3_ref, o_ref)


def _finalize_slice(prev_out, s_k, edge_feats, w3t, k):
    blk0 = k * _BLK_PER_SLICE
    common = dict(
        grid=(_BLK_PER_SLICE,),
        out_specs=pl.BlockSpec((_BLK, HIDDEN), lambda i, b0=blk0: (b0 + i, 0)),
        out_shape=jax.ShapeDtypeStruct((N_EDGES, HIDDEN), jnp.float32),
    )
    data_specs = [
        pl.BlockSpec((_BLK // 2, HIDDEN), lambda i: (i, 0)),
        pl.BlockSpec((_BLK, D_EDGE), lambda i, b0=blk0: (b0 + i, 0)),
        pl.BlockSpec((D_EDGE, HIDDEN), lambda i: (0, 0)),
    ]
    if prev_out is None:
        return pl.pallas_call(
            _final_body0, in_specs=data_specs, **common,
        )(s_k, edge_feats, w3t)
    return pl.pallas_call(
        _final_body,
        in_specs=[pl.BlockSpec((8, HIDDEN), lambda i: (0, 0))] + data_specs,
        input_output_aliases={0: 0},
        **common,
    )(prev_out, s_k, edge_feats, w3t)


def kernel(node_feats, edge_feats, global_feats, edge_index, batch, W):
    wt = W.T
    w1t = wt[:D_NODE]
    w2t = wt[D_NODE:2 * D_NODE]
    w3t = wt[2 * D_NODE:2 * D_NODE + D_EDGE]
    w4t = wt[2 * D_NODE + D_EDGE:]
    batch_f = batch.astype(jnp.int32)[:, None]
    src = edge_index[0].astype(jnp.int32)
    dst = edge_index[1].astype(jnp.int32)

    a_tbl, b_tbl = _build_tables(node_feats, batch_f, global_feats,
                                 w1t, w2t, w4t)

    src5 = src.reshape(_NSLICE, _NW, _NOUTER, _NSLOT, _CHUNK)
    dst5 = dst.reshape(_NSLICE, _NW, _NOUTER, _NSLOT, _CHUNK)
    s_slices = [_gather_add(a_tbl, b_tbl, src5[k], dst5[k])
                for k in range(_NSLICE)]

    out = None
    for k in range(_NSLICE):
        out = _finalize_slice(out, s_slices[k], edge_feats, w3t, k)
    return out
